# baseline (speedup 1.0000x reference)
import functools

import jax
import jax.numpy as jnp
from jax import lax
from jax.experimental import pallas as pl
from jax.experimental.pallas import tpu as pltpu
from jax.experimental.pallas import tpu_sc as plsc

NC = 2
NS = 16
NW = NC * NS
L = 16
CHUNK = 128


def _sc_accumulate(NCH, V_pad, STRIPE, T_pad, TSTRIPE):
    mesh = plsc.VectorSubcoreMesh(
        core_axis_name="c", subcore_axis_name="s", num_cores=NC, num_subcores=NS
    )

    @functools.partial(
        pl.kernel,
        out_type=jax.ShapeDtypeStruct((NC, 6, V_pad), jnp.float32),
        mesh=mesh,
        scratch_types=[
            [pltpu.VMEM((NCH, CHUNK), jnp.int32) for _ in range(6)],
            [[pltpu.VMEM((CHUNK,), jnp.float32) for _ in range(9)]
             for _ in range(2)],
            [[pltpu.VMEM((CHUNK,), jnp.float32) for _ in range(6)]
             for _ in range(2)],
            [[pltpu.VMEM((CHUNK,), jnp.float32) for _ in range(6)]
             for _ in range(2)],
            [pltpu.VMEM_SHARED((V_pad,), jnp.float32) for _ in range(3)],
            [pltpu.VMEM_SHARED((T_pad,), jnp.float32) for _ in range(2)],
            [pltpu.VMEM_SHARED((V_pad,), jnp.float32) for _ in range(6)],
            pltpu.VMEM((max(STRIPE, TSTRIPE),), jnp.float32),
            pltpu.SemaphoreType.DMA,
            [pltpu.SemaphoreType.DMA for _ in range(2)],
        ],
    )
    def k(px, py, pz, tu, tv, ih, jh, zeros_hbm, out_hbm,
          idxv, gpsets, gtsets, cbsets, psh, tsh, accs, bounce, semg, sems):
        cid = lax.axis_index("c")
        sid = lax.axis_index("s")
        wid = cid * NS + sid

        for a in accs:
            pltpu.sync_copy(zeros_hbm.at[pl.ds(0, STRIPE)],
                            a.at[pl.ds(sid * STRIPE, STRIPE)])

        vst_ = pl.ds(sid * STRIPE, STRIPE)
        tst_ = pl.ds(sid * TSTRIPE, TSTRIPE)
        vb = bounce.at[pl.ds(0, STRIPE)]
        tb = bounce.at[pl.ds(0, TSTRIPE)]
        for hbm, sh in ((px, psh[0]), (py, psh[1]), (pz, psh[2])):
            pltpu.sync_copy(hbm.at[vst_], vb)
            pltpu.sync_copy(vb, sh.at[vst_])
        for hbm, sh in ((tu, tsh[0]), (tv, tsh[1])):
            pltpu.sync_copy(hbm.at[tst_], tb)
            pltpu.sync_copy(tb, sh.at[tst_])

        for corner in range(3):
            pltpu.sync_copy(ih.at[corner, wid], idxv[corner])
            pltpu.sync_copy(jh.at[corner, wid], idxv[3 + corner])

        plsc.subcore_barrier()

        i0v, i1v, i2v, j0v, j1v, j2v = idxv

        def gather_ops(jc, b, op):
            gp, gt = gpsets[b], gtsets[b]
            for corner, iv in enumerate((i0v, i1v, i2v)):
                idx = iv.at[jc]
                for comp in range(3):
                    op(psh[comp].at[idx], gp[corner * 3 + comp], semg)
            for corner, jv in enumerate((j0v, j1v, j2v)):
                idx = jv.at[jc]
                for comp in range(2):
                    op(tsh[comp].at[idx], gt[corner * 2 + comp], semg)

        def scatter_ops(jc, b, op):
            cb = cbsets[b]
            for iv in (i0v, i1v, i2v):
                idx = iv.at[jc]
                for comp in range(6):
                    op(cb[comp], accs[comp].at[idx], sems[b])

        def compute(b):
            gp, gt = gpsets[b], gtsets[b]
            cb = cbsets[b]
            for s in range(CHUNK // L):
                sl = pl.ds(s * L, L)
                x0, y0, z0 = gp[0][sl], gp[1][sl], gp[2][sl]
                x1, y1, z1 = gp[3][sl], gp[4][sl], gp[5][sl]
                x2, y2, z2 = gp[6][sl], gp[7][sl], gp[8][sl]
                u0, v0 = gt[0][sl], gt[1][sl]
                u1, v1 = gt[2][sl], gt[3][sl]
                u2, v2 = gt[4][sl], gt[5][sl]
                e1x = x1 - x0
                e1y = y1 - y0
                e1z = z1 - z0
                e2x = x2 - x0
                e2y = y2 - y0
                e2z = z2 - z0
                du1 = u1 - u0
                dv1 = v1 - v0
                du2 = u2 - u0
                dv2 = v2 - v0
                den = du1 * dv2 - dv1 * du2
                dens = jnp.where(
                    den > 0.0, jnp.maximum(den, 1e-6), jnp.minimum(den, -1e-6)
                )
                inv = 1.0 / dens
                cb[0][sl] = e1y * e2z - e1z * e2y
                cb[1][sl] = e1z * e2x - e1x * e2z
                cb[2][sl] = e1x * e2y - e1y * e2x
                cb[3][sl] = (e1x * dv2 - e2x * dv1) * inv
                cb[4][sl] = (e1y * dv2 - e2y * dv1) * inv
                cb[5][sl] = (e1z * dv2 - e2z * dv1) * inv

        def issue(s, d, m):
            pltpu.async_copy(s, d, m)

        def issue_add(s, d, m):
            pltpu.async_copy(s, d, m, add=True)

        def drain_gathers():
            pltpu.make_async_copy(
                zeros_hbm.at[pl.ds(0, 15 * CHUNK)],
                bounce.at[pl.ds(0, 15 * CHUNK)], semg).wait()

        def drain_scatters(b):
            pltpu.make_async_copy(
                zeros_hbm.at[pl.ds(0, 18 * CHUNK)],
                bounce.at[pl.ds(0, 18 * CHUNK)], sems[b]).wait()

        gather_ops(0, 0, issue)

        @pl.loop(0, NCH, step=2)
        def _(base):
            for b in range(2):
                jc = base + b
                drain_gathers()

                @pl.when(jc + 1 < NCH)
                def _():
                    gather_ops(jc + 1, 1 - b, issue)

                @pl.when(jc >= 2)
                def _():
                    drain_scatters(b)

                compute(b)
                scatter_ops(jc, b, issue_add)

        for b in range(2):
            drain_scatters(b)

        plsc.subcore_barrier()

        st = pl.ds(sid * STRIPE, STRIPE)
        for comp in range(6):
            pltpu.sync_copy(accs[comp].at[st], out_hbm.at[cid, comp, st])

    return k


def _tc_finalize(partial, V_pad, V, VB=2048):
    def body(p_ref, o_ref):
        p = p_ref[...]
        a = p[0] + p[1]
        nx, ny, nz = a[0], a[1], a[2]
        tx, ty, tz = a[3], a[4], a[5]
        nd = nx * nx + ny * ny + nz * nz
        cond = nd > 1e-20
        nx = jnp.where(cond, nx, 0.0)
        ny = jnp.where(cond, ny, 0.0)
        nz = jnp.where(cond, nz, 1.0)
        rinv = 1.0 / jnp.sqrt(jnp.maximum(nx * nx + ny * ny + nz * nz, 1e-20))
        nx, ny, nz = nx * rinv, ny * rinv, nz * rinv
        tinv = 1.0 / jnp.sqrt(jnp.maximum(tx * tx + ty * ty + tz * tz, 1e-20))
        tx, ty, tz = tx * tinv, ty * tinv, tz * tinv
        d = tx * nx + ty * ny + tz * nz
        tx, ty, tz = tx - d * nx, ty - d * ny, tz - d * nz
        tinv = 1.0 / jnp.sqrt(jnp.maximum(tx * tx + ty * ty + tz * tz, 1e-20))
        tx, ty, tz = tx * tinv, ty * tinv, tz * tinv
        o_ref[...] = jnp.stack(
            [jnp.stack([nx, ny, nz], axis=-1),
             jnp.stack([tx, ty, tz], axis=-1)]
        )

    return pl.pallas_call(
        body,
        grid=(V_pad // VB,),
        in_specs=[pl.BlockSpec((NC, 6, VB), lambda i: (0, 0, i))],
        out_specs=pl.BlockSpec((2, VB, 3), lambda i: (0, i, 0)),
        out_shape=jax.ShapeDtypeStruct((2, V, 3), jnp.float32),
    )(partial)


def kernel(positions, texcoords, pos_indexes, uv_indexes, mvp):
    V = positions.shape[0]
    T = texcoords.shape[0]
    F = pos_indexes.shape[0]

    per_w = -(-F // NW)
    NCH = -(-per_w // CHUNK)
    NCH += NCH % 2
    F_pad = NW * NCH * CHUNK
    V_pad = (V // (NS * CHUNK) + 1) * (NS * CHUNK)
    STRIPE = V_pad // NS
    T_pad = -(-T // CHUNK) * CHUNK
    TSTRIPE = T_pad // NS

    ppad = jnp.pad(positions, ((0, V_pad - V), (0, 0)))
    px, py, pz = ppad[:, 0], ppad[:, 1], ppad[:, 2]
    tpad = jnp.pad(texcoords, ((0, T_pad - T), (0, 0)))
    tu, tv = tpad[:, 0], tpad[:, 1]
    ih = jnp.pad(pos_indexes.T, ((0, 0), (0, F_pad - F)),
                 constant_values=V).reshape(3, NW, NCH, CHUNK)
    jh = jnp.pad(uv_indexes.T, ((0, 0), (0, F_pad - F)),
                 constant_values=0).reshape(3, NW, NCH, CHUNK)
    zrows = jnp.zeros((max(STRIPE, TSTRIPE),), jnp.float32)

    partial = _sc_accumulate(NCH, V_pad, STRIPE, T_pad, TSTRIPE)(
        px, py, pz, tu, tv, ih, jh, zrows
    )
    return _tc_finalize(partial, V_pad, V)


# speedup vs baseline: 1.5077x; 1.5077x over previous
import functools

import jax
import jax.numpy as jnp
from jax import lax
from jax.experimental import pallas as pl
from jax.experimental.pallas import tpu as pltpu
from jax.experimental.pallas import tpu_sc as plsc

NC = 2
NS = 16
NW = NC * NS
L = 16
CHUNK = 128


def _sc_accumulate(NCH, V_pad, STRIPE, T_pad, TSTRIPE):
    mesh = plsc.VectorSubcoreMesh(
        core_axis_name="c", subcore_axis_name="s", num_cores=NC, num_subcores=NS
    )

    @functools.partial(
        pl.kernel,
        out_type=jax.ShapeDtypeStruct((NC, 6, V_pad), jnp.float32),
        mesh=mesh,
        scratch_types=[
            [pltpu.VMEM((NCH, CHUNK), jnp.int32) for _ in range(6)],
            [[pltpu.VMEM((CHUNK,), jnp.float32) for _ in range(9)]
             for _ in range(2)],
            [[pltpu.VMEM((CHUNK,), jnp.float32) for _ in range(6)]
             for _ in range(2)],
            [[pltpu.VMEM((CHUNK,), jnp.float32) for _ in range(6)]
             for _ in range(2)],
            [pltpu.VMEM_SHARED((V_pad,), jnp.float32) for _ in range(3)],
            [pltpu.VMEM_SHARED((T_pad,), jnp.float32) for _ in range(2)],
            [pltpu.VMEM_SHARED((V_pad,), jnp.float32) for _ in range(6)],
            pltpu.VMEM((max(STRIPE, TSTRIPE),), jnp.float32),
            pltpu.SemaphoreType.DMA,
            [pltpu.SemaphoreType.DMA for _ in range(2)],
        ],
    )
    def k(px, py, pz, tu, tv, ih, jh, zeros_hbm, out_hbm,
          idxv, gpsets, gtsets, cbsets, psh, tsh, accs, bounce, semg, sems):
        cid = lax.axis_index("c")
        sid = lax.axis_index("s")
        wid = cid * NS + sid

        zcps = [
            pltpu.async_copy(zeros_hbm.at[pl.ds(0, STRIPE)],
                             a.at[pl.ds(sid * STRIPE, STRIPE)], sems[0])
            for a in accs
        ]
        for corner in range(3):
            zcps.append(pltpu.async_copy(ih.at[corner, wid], idxv[corner],
                                         sems[0]))
            zcps.append(pltpu.async_copy(jh.at[corner, wid],
                                         idxv[3 + corner], sems[0]))

        vst_ = pl.ds(sid * STRIPE, STRIPE)
        tst_ = pl.ds(sid * TSTRIPE, TSTRIPE)
        vb = bounce.at[pl.ds(0, STRIPE)]
        tb = bounce.at[pl.ds(0, TSTRIPE)]
        for hbm, sh in ((px, psh[0]), (py, psh[1]), (pz, psh[2])):
            pltpu.sync_copy(hbm.at[vst_], vb)
            pltpu.sync_copy(vb, sh.at[vst_])
        for hbm, sh in ((tu, tsh[0]), (tv, tsh[1])):
            pltpu.sync_copy(hbm.at[tst_], tb)
            pltpu.sync_copy(tb, sh.at[tst_])

        for cp in zcps:
            cp.wait()

        plsc.subcore_barrier()

        i0v, i1v, i2v, j0v, j1v, j2v = idxv

        def gather_ops(jc, b, op):
            gp, gt = gpsets[b], gtsets[b]
            for corner, iv in enumerate((i0v, i1v, i2v)):
                idx = iv.at[jc]
                for comp in range(3):
                    op(psh[comp].at[idx], gp[corner * 3 + comp], semg)
            for corner, jv in enumerate((j0v, j1v, j2v)):
                idx = jv.at[jc]
                for comp in range(2):
                    op(tsh[comp].at[idx], gt[corner * 2 + comp], semg)

        def scatter_ops(jc, b, op):
            cb = cbsets[b]
            for iv in (i0v, i1v, i2v):
                idx = iv.at[jc]
                for comp in range(6):
                    op(cb[comp], accs[comp].at[idx], sems[b])

        def compute(b):
            gp, gt = gpsets[b], gtsets[b]
            cb = cbsets[b]
            for s in range(CHUNK // L):
                sl = pl.ds(s * L, L)
                x0, y0, z0 = gp[0][sl], gp[1][sl], gp[2][sl]
                x1, y1, z1 = gp[3][sl], gp[4][sl], gp[5][sl]
                x2, y2, z2 = gp[6][sl], gp[7][sl], gp[8][sl]
                u0, v0 = gt[0][sl], gt[1][sl]
                u1, v1 = gt[2][sl], gt[3][sl]
                u2, v2 = gt[4][sl], gt[5][sl]
                e1x = x1 - x0
                e1y = y1 - y0
                e1z = z1 - z0
                e2x = x2 - x0
                e2y = y2 - y0
                e2z = z2 - z0
                du1 = u1 - u0
                dv1 = v1 - v0
                du2 = u2 - u0
                dv2 = v2 - v0
                den = du1 * dv2 - dv1 * du2
                dens = jnp.where(
                    den > 0.0, jnp.maximum(den, 1e-6), jnp.minimum(den, -1e-6)
                )
                inv = 1.0 / dens
                cb[0][sl] = e1y * e2z - e1z * e2y
                cb[1][sl] = e1z * e2x - e1x * e2z
                cb[2][sl] = e1x * e2y - e1y * e2x
                cb[3][sl] = (e1x * dv2 - e2x * dv1) * inv
                cb[4][sl] = (e1y * dv2 - e2y * dv1) * inv
                cb[5][sl] = (e1z * dv2 - e2z * dv1) * inv

        def issue(s, d, m):
            pltpu.async_copy(s, d, m)

        def issue_add(s, d, m):
            pltpu.async_copy(s, d, m, add=True)

        def drain_gathers():
            pltpu.make_async_copy(
                zeros_hbm.at[pl.ds(0, 15 * CHUNK)],
                bounce.at[pl.ds(0, 15 * CHUNK)], semg).wait()

        def drain_scatters(b):
            pltpu.make_async_copy(
                zeros_hbm.at[pl.ds(0, 18 * CHUNK)],
                bounce.at[pl.ds(0, 18 * CHUNK)], sems[b]).wait()

        gather_ops(0, 0, issue)

        @pl.loop(0, NCH, step=2)
        def _(base):
            for b in range(2):
                jc = base + b
                drain_gathers()

                @pl.when(jc + 1 < NCH)
                def _():
                    gather_ops(jc + 1, 1 - b, issue)

                @pl.when(jc >= 2)
                def _():
                    drain_scatters(b)

                compute(b)
                scatter_ops(jc, b, issue_add)

        for b in range(2):
            drain_scatters(b)

        plsc.subcore_barrier()

        st = pl.ds(sid * STRIPE, STRIPE)
        wcps = [
            pltpu.async_copy(accs[comp].at[st], out_hbm.at[cid, comp, st],
                             sems[0])
            for comp in range(6)
        ]
        for cp in wcps:
            cp.wait()

    return k


def _tc_finalize(partial, V_pad, VB=2048):
    def body(p_ref, o_ref):
        p = p_ref[...]
        a = p[0] + p[1]
        nx, ny, nz = a[0], a[1], a[2]
        tx, ty, tz = a[3], a[4], a[5]
        nd = nx * nx + ny * ny + nz * nz
        cond = nd > 1e-20
        nx = jnp.where(cond, nx, 0.0)
        ny = jnp.where(cond, ny, 0.0)
        nz = jnp.where(cond, nz, 1.0)
        rinv = 1.0 / jnp.sqrt(jnp.maximum(nx * nx + ny * ny + nz * nz, 1e-20))
        nx, ny, nz = nx * rinv, ny * rinv, nz * rinv
        tinv = 1.0 / jnp.sqrt(jnp.maximum(tx * tx + ty * ty + tz * tz, 1e-20))
        tx, ty, tz = tx * tinv, ty * tinv, tz * tinv
        d = tx * nx + ty * ny + tz * nz
        tx, ty, tz = tx - d * nx, ty - d * ny, tz - d * nz
        tinv = 1.0 / jnp.sqrt(jnp.maximum(tx * tx + ty * ty + tz * tz, 1e-20))
        tx, ty, tz = tx * tinv, ty * tinv, tz * tinv
        zr = jnp.zeros_like(nx)
        o_ref[...] = jnp.stack(
            [jnp.stack([nx, ny, nz, zr]), jnp.stack([tx, ty, tz, zr])]
        )

    return pl.pallas_call(
        body,
        grid=(V_pad // VB,),
        in_specs=[pl.BlockSpec((NC, 6, VB), lambda i: (0, 0, i))],
        out_specs=pl.BlockSpec((2, 4, VB), lambda i: (0, 0, i)),
        out_shape=jax.ShapeDtypeStruct((2, 4, V_pad), jnp.float32),
    )(partial)


def kernel(positions, texcoords, pos_indexes, uv_indexes, mvp):
    V = positions.shape[0]
    T = texcoords.shape[0]
    F = pos_indexes.shape[0]

    per_w = -(-F // NW)
    NCH = -(-per_w // CHUNK)
    NCH += NCH % 2
    F_pad = NW * NCH * CHUNK
    V_pad = (V // (NS * CHUNK) + 1) * (NS * CHUNK)
    STRIPE = V_pad // NS
    T_pad = -(-T // CHUNK) * CHUNK
    TSTRIPE = T_pad // NS

    ppad = jnp.pad(positions, ((0, V_pad - V), (0, 0)))
    px, py, pz = ppad[:, 0], ppad[:, 1], ppad[:, 2]
    tpad = jnp.pad(texcoords, ((0, T_pad - T), (0, 0)))
    tu, tv = tpad[:, 0], tpad[:, 1]
    ih = jnp.pad(pos_indexes.T, ((0, 0), (0, F_pad - F)),
                 constant_values=V).reshape(3, NW, NCH, CHUNK)
    jh = jnp.pad(uv_indexes.T, ((0, 0), (0, F_pad - F)),
                 constant_values=0).reshape(3, NW, NCH, CHUNK)
    zrows = jnp.zeros((max(STRIPE, TSTRIPE),), jnp.float32)

    partial = _sc_accumulate(NCH, V_pad, STRIPE, T_pad, TSTRIPE)(
        px, py, pz, tu, tv, ih, jh, zrows
    )
    out = _tc_finalize(partial, V_pad)
    return jnp.transpose(out[:, :3, :V], (0, 2, 1))
